# baseline (speedup 1.0000x reference)
import jax
import jax.numpy as jnp
from jax import lax
from jax.experimental import pallas as pl
from jax.experimental.pallas import tpu as pltpu
from jax.experimental.pallas import tpu_sc as plsc

_N = 10000
_E = 320000
_D = 128
_HD = 64
_NC = 2
_NS = 16
_NW = _NC * _NS
_CH = 125
_EPT = _E // _NS
_NCHUNK = _EPT // _CH
_NPAD = 10240
_RPT = _NPAD // _NS
_NROWS = _N // _NS
_G = 16
_NGRP = _NCHUNK // _G
_EPT32 = _E // _NW
_NCH32 = _EPT32 // _CH


def _seg_body(y_hbm, src_hbm, dst_hbm, agg_out, srcA, dstA, srcB, dstB,
              rows0, rows1, ysp, acc, sem0, sem1, semi):
    c = lax.axis_index("c")
    s = lax.axis_index("s")

    pltpu.sync_copy(y_hbm.at[c, pl.ds(s * _NROWS, _NROWS)],
                    ysp.at[pl.ds(s * _NROWS, _NROWS)])

    def _zrow(i, _):
        for k in range(_HD // 16):
            rows0[i, pl.ds(k * 16, 16)] = jnp.zeros((16,), jnp.float32)
        return 0
    lax.fori_loop(0, _CH, _zrow, 0)
    base = s * _RPT
    for t in range(5):
        pltpu.sync_copy(rows0, acc.at[pl.ds(base + t * _CH, _CH)])
    pltpu.sync_copy(rows0.at[pl.ds(0, _RPT - 5 * _CH)],
                    acc.at[pl.ds(base + 5 * _CH, _RPT - 5 * _CH)])
    plsc.subcore_barrier()

    idx = [(srcA, dstA), (srcB, dstB)]
    pltpu.sync_copy(src_hbm.at[s, pl.ds(0, _G)], srcA)
    pltpu.sync_copy(dst_hbm.at[s, pl.ds(0, _G)], dstA)
    pltpu.async_copy(ysp.at[srcA.at[0]], rows0, sem0)
    pltpu.async_copy(ysp.at[srcA.at[1]], rows1, sem1)
    for g in range(_NGRP):
        srcg, dstg = idx[g % 2]
        srcn, dstn = idx[(g + 1) % 2]
        if g + 1 < _NGRP:
            cpi0 = pltpu.async_copy(src_hbm.at[s, pl.ds((g + 1) * _G, _G)],
                                    srcn, semi)
            cpi1 = pltpu.async_copy(dst_hbm.at[s, pl.ds((g + 1) * _G, _G)],
                                    dstn, semi)

        def _step(i, _):
            j0 = i * 2
            cp0 = pltpu.make_async_copy(ysp.at[srcg.at[j0]], rows0, sem0)
            cp1 = pltpu.make_async_copy(ysp.at[srcg.at[j0 + 1]], rows1,
                                        sem1)
            cp0.wait()
            pltpu.sync_copy(rows0, acc.at[dstg.at[j0]], add=True)
            pltpu.async_copy(ysp.at[srcg.at[j0 + 2]], rows0, sem0)
            cp1.wait()
            pltpu.sync_copy(rows1, acc.at[dstg.at[j0 + 1]], add=True)
            pltpu.async_copy(ysp.at[srcg.at[j0 + 3]], rows1, sem1)
            return 0
        lax.fori_loop(0, _G // 2 - 1, _step, 0)

        if g + 1 < _NGRP:
            cpi0.wait()
            cpi1.wait()
        pltpu.make_async_copy(ysp.at[srcg.at[_G - 2]], rows0, sem0).wait()
        pltpu.sync_copy(rows0, acc.at[dstg.at[_G - 2]], add=True)
        if g + 1 < _NGRP:
            pltpu.async_copy(ysp.at[srcn.at[0]], rows0, sem0)
        pltpu.make_async_copy(ysp.at[srcg.at[_G - 1]], rows1, sem1).wait()
        pltpu.sync_copy(rows1, acc.at[dstg.at[_G - 1]], add=True)
        if g + 1 < _NGRP:
            pltpu.async_copy(ysp.at[srcn.at[1]], rows1, sem1)

    plsc.subcore_barrier()

    for t in range(5):
        r = base + t * _CH
        pltpu.sync_copy(acc.at[pl.ds(r, _CH)], rows0)
        pltpu.sync_copy(rows0, agg_out.at[c, pl.ds(r, _CH)])
    rem = _RPT - 5 * _CH
    pltpu.sync_copy(acc.at[pl.ds(base + 5 * _CH, rem)],
                    rows0.at[pl.ds(0, rem)])
    pltpu.sync_copy(rows0.at[pl.ds(0, rem)],
                    agg_out.at[c, pl.ds(base + 5 * _CH, rem)])


def _deg_body(dst_hbm, deg_out, dstv, ones, dacc):
    c = lax.axis_index("c")
    s = lax.axis_index("s")
    blk = c * _NS + s

    pltpu.sync_copy(dst_hbm.at[blk], dstv)

    def _fill(val):
        def _row(i, _):
            ones[i, :] = jnp.full((16,), val, jnp.float32)
            return 0
        lax.fori_loop(0, 128, _row, 0)
    _fill(0.0)
    for t in range(_RPT // 128):
        pltpu.sync_copy(ones, dacc.at[pl.ds(s * _RPT + t * 128, 128)])
    _fill(1.0)
    plsc.subcore_barrier()

    o = ones.at[pl.ds(0, _CH)]
    def _step(j, _):
        pltpu.sync_copy(o, dacc.at[dstv.at[j]], add=True)
        return 0
    lax.fori_loop(0, _NCH32, _step, 0)

    plsc.subcore_barrier()

    for t in range(_RPT // 128):
        r = s * _RPT + t * 128
        pltpu.sync_copy(dacc.at[pl.ds(r, 128)], ones)
        pltpu.sync_copy(ones, deg_out.at[c, pl.ds(r, 128)])


def _make_seg():
    mesh = plsc.VectorSubcoreMesh(core_axis_name="c", subcore_axis_name="s")
    return pl.kernel(
        _seg_body,
        out_type=[jax.ShapeDtypeStruct((_NC, _NPAD, _HD), jnp.float32)],
        mesh=mesh,
        scratch_types=[
            pltpu.VMEM((_G, _CH), jnp.int32),
            pltpu.VMEM((_G, _CH), jnp.int32),
            pltpu.VMEM((_G, _CH), jnp.int32),
            pltpu.VMEM((_G, _CH), jnp.int32),
            pltpu.VMEM((_CH, _HD), jnp.float32),
            pltpu.VMEM((_CH, _HD), jnp.float32),
            pltpu.VMEM_SHARED((_N, _HD), jnp.float32),
            pltpu.VMEM_SHARED((_NPAD, _HD), jnp.float32),
            pltpu.SemaphoreType.DMA,
            pltpu.SemaphoreType.DMA,
            pltpu.SemaphoreType.DMA,
        ],
        compiler_params=pltpu.CompilerParams(use_tc_tiling_on_sc=False),
        name="seg_sum",
    )


def _make_deg():
    mesh = plsc.VectorSubcoreMesh(core_axis_name="c", subcore_axis_name="s")
    return pl.kernel(
        _deg_body,
        out_type=[jax.ShapeDtypeStruct((_NC, _NPAD, 16), jnp.float32)],
        mesh=mesh,
        scratch_types=[
            pltpu.VMEM((_NCH32, _CH), jnp.int32),
            pltpu.VMEM((128, 16), jnp.float32),
            pltpu.VMEM_SHARED((_NPAD, 16), jnp.float32),
        ],
        compiler_params=pltpu.CompilerParams(use_tc_tiling_on_sc=False),
        name="deg_sum",
    )


_seg = _make_seg()
_deg = _make_deg()


def _bn(h, g, be):
    m = jnp.mean(h, axis=0, keepdims=True)
    v = jnp.mean((h - m) * (h - m), axis=0, keepdims=True)
    return (h - m) / jnp.sqrt(v + 1e-5) * g + be


def _deg_from(dp_ref):
    deg = dp_ref[0, : _N, 0:1] + dp_ref[1, : _N, 0:1]
    return jnp.maximum(deg, 1.0)


def _halves(sp_ref):
    return jnp.concatenate([sp_ref[0, : _N, :], sp_ref[1, : _N, :]], axis=1)


def _tc1_body(sp_ref, dp_ref, x_ref, wl1t_ref, wr1t_ref, b1_ref, g1_ref,
              be1_ref, h1_ref):
    agg = _halves(sp_ref) / _deg_from(dp_ref)
    pre = (jnp.dot(agg, wl1t_ref[...], preferred_element_type=jnp.float32)
           + b1_ref[...]
           + jnp.dot(x_ref[...], wr1t_ref[...],
                     preferred_element_type=jnp.float32))
    h = jnp.maximum(pre, 0.0)
    h1 = _bn(h, g1_ref[...], be1_ref[...])
    h1_ref[0, :, :] = h1[:, : _HD]
    h1_ref[1, :, :] = h1[:, _HD:]


def _tc2_body(sp_ref, dp_ref, h1_ref, wl2t_ref, wr2t_ref, b2_ref, g2_ref,
              be2_ref, wl3t_ref, h2_ref, y3_ref):
    agg = _halves(sp_ref) / _deg_from(dp_ref)
    h1 = jnp.concatenate([h1_ref[0, :, :], h1_ref[1, :, :]], axis=1)
    pre = (jnp.dot(agg, wl2t_ref[...], preferred_element_type=jnp.float32)
           + b2_ref[...]
           + jnp.dot(h1, wr2t_ref[...], preferred_element_type=jnp.float32))
    h = jnp.maximum(pre, 0.0)
    h2 = _bn(h, g2_ref[...], be2_ref[...])
    h2_ref[...] = h2
    y3 = jnp.dot(h2, wl3t_ref[...], preferred_element_type=jnp.float32)
    y3_ref[0, :, :] = y3[:, : _HD]
    y3_ref[1, :, :] = y3[:, _HD:]


def _tc3_body(sp_ref, dp_ref, h2_ref, wr3t_ref, b3_ref, eps_ref, z_ref):
    pre = (_halves(sp_ref) / _deg_from(dp_ref) + b3_ref[...]
           + jnp.dot(h2_ref[...], wr3t_ref[...],
                     preferred_element_type=jnp.float32))
    mean = pre[:, : _HD]
    log_std = pre[:, _HD:]
    z_ref[...] = mean + jnp.exp(log_std) * eps_ref[...]


_tc1 = pl.pallas_call(
    _tc1_body, out_shape=jax.ShapeDtypeStruct((_NC, _N, _HD), jnp.float32))
_tc2 = pl.pallas_call(
    _tc2_body, out_shape=[jax.ShapeDtypeStruct((_N, 256), jnp.float32),
                          jax.ShapeDtypeStruct((_NC, _N, _HD), jnp.float32)],
    compiler_params=pltpu.CompilerParams(
        vmem_limit_bytes=100 * 1024 * 1024))
_tc3 = pl.pallas_call(
    _tc3_body, out_shape=jax.ShapeDtypeStruct((_N, _HD), jnp.float32))


def kernel(x, edge_index, Wl1, Wr1, b1, g1, be1, Wl2, Wr2, b2, g2, be2,
           Wl3, Wr3, b3, eps):
    src = edge_index[0].reshape(_NS, _NCHUNK, _CH)
    dst = edge_index[1].reshape(_NS, _NCHUNK, _CH)
    dst32 = edge_index[1].reshape(_NW, _NCH32, _CH)
    x2 = x.reshape(_N, _NC, _HD).swapaxes(0, 1)

    (d1,) = _deg(dst32)
    (s1,) = _seg(x2, src, dst)
    h1 = _tc1(s1, d1, x, Wl1.T, Wr1.T, b1[None, :], g1[None, :],
              be1[None, :])
    (s2,) = _seg(h1, src, dst)
    h2, y3 = _tc2(s2, d1, h1, Wl2.T, Wr2.T, b2[None, :], g2[None, :],
                  be2[None, :], Wl3.T)
    (s3,) = _seg(y3, src, dst)
    z = _tc3(s3, d1, h2, Wr3.T, b3[None, :], eps)
    return z


# speedup vs baseline: 1.3531x; 1.3531x over previous
import functools

import jax
import jax.numpy as jnp
from jax import lax
from jax.experimental import pallas as pl
from jax.experimental.pallas import tpu as pltpu
from jax.experimental.pallas import tpu_sc as plsc

_N = 10000
_E = 320000
_D = 128
_NC = 2
_NS = 16
_NW = _NC * _NS
_EPT = _E // _NW
_CH = 80
_NCHUNK = _EPT // _CH
_NPAD = 10240
_RPT = _NPAD // _NS
_NBUF = 4
_G = 25
_NGRP = _NCHUNK // _G
_DCH = 125
_NCH32 = _EPT // _DCH


def _seg_body(y_hbm, src_hbm, dst_hbm, agg_out, srcA, dstA, srcB, dstB,
              rows0, rows1, rows2, rows3, acc, semg0, semg1, semg2, semg3,
              sems0, sems1, sems2, sems3, semi):
    c = lax.axis_index("c")
    s = lax.axis_index("s")
    blk = c * _NS + s
    rows = [rows0, rows1, rows2, rows3]
    semg = [semg0, semg1, semg2, semg3]
    sems = [sems0, sems1, sems2, sems3]
    idx = [(srcA, dstA), (srcB, dstB)]

    def _zrow(i, _):
        for k in range(8):
            rows0[i, pl.ds(k * 16, 16)] = jnp.zeros((16,), jnp.float32)
        return 0
    lax.fori_loop(0, _CH, _zrow, 0)
    base = s * _RPT
    for t in range(_RPT // _CH):
        pltpu.sync_copy(rows0, acc.at[pl.ds(base + t * _CH, _CH)])
    plsc.subcore_barrier()

    pltpu.sync_copy(src_hbm.at[blk, pl.ds(0, _G)], srcA)
    pltpu.sync_copy(dst_hbm.at[blk, pl.ds(0, _G)], dstA)
    cpg = [None] * _NCHUNK
    cps = [None] * _NCHUNK
    cpi = None

    def _gather(j):
        g, r = divmod(j, _G)
        srcg = idx[g % 2][0]
        cpg[j] = pltpu.async_copy(y_hbm.at[srcg.at[r]], rows[j % _NBUF],
                                  semg[j % _NBUF])

    _gather(0)
    _gather(1)
    for j in range(_NCHUNK):
        b = j % _NBUF
        g, r = divmod(j, _G)
        if r == 2 and g + 1 < _NGRP:
            srcn, dstn = idx[(g + 1) % 2]
            cpi = (pltpu.async_copy(src_hbm.at[blk, pl.ds((g + 1) * _G, _G)],
                                    srcn, semi),
                   pltpu.async_copy(dst_hbm.at[blk, pl.ds((g + 1) * _G, _G)],
                                    dstn, semi))
        if r == _G - 3 and g + 1 < _NGRP:
            cpi[0].wait()
            cpi[1].wait()
        cpg[j].wait()
        dstg = idx[g % 2][1]
        cps[j] = pltpu.async_copy(rows[b], acc.at[dstg.at[r]], sems[b],
                                  add=True)
        if j >= 2:
            cps[j - 2].wait()
        if j + 2 < _NCHUNK:
            _gather(j + 2)
    cps[_NCHUNK - 2].wait()
    cps[_NCHUNK - 1].wait()

    plsc.subcore_barrier()

    for t in range(_RPT // _CH):
        r = base + t * _CH
        pltpu.sync_copy(acc.at[pl.ds(r, _CH)], rows0)
        pltpu.sync_copy(rows0, agg_out.at[c, pl.ds(r, _CH)])


def _deg_body(dst_hbm, deg_out, dstv, ones, dacc):
    c = lax.axis_index("c")
    s = lax.axis_index("s")
    blk = c * _NS + s

    pltpu.sync_copy(dst_hbm.at[blk], dstv)

    def _fill(val):
        def _row(i, _):
            ones[i, :] = jnp.full((16,), val, jnp.float32)
            return 0
        lax.fori_loop(0, 128, _row, 0)
    _fill(0.0)
    for t in range(_RPT // 128):
        pltpu.sync_copy(ones, dacc.at[pl.ds(s * _RPT + t * 128, 128)])
    _fill(1.0)
    plsc.subcore_barrier()

    o = ones.at[pl.ds(0, _DCH)]
    def _step(j, _):
        pltpu.sync_copy(o, dacc.at[dstv.at[j]], add=True)
        return 0
    lax.fori_loop(0, _NCH32, _step, 0)

    plsc.subcore_barrier()

    for t in range(_RPT // 128):
        r = s * _RPT + t * 128
        pltpu.sync_copy(dacc.at[pl.ds(r, 128)], ones)
        pltpu.sync_copy(ones, deg_out.at[c, pl.ds(r, 128)])


def _make_seg():
    mesh = plsc.VectorSubcoreMesh(core_axis_name="c", subcore_axis_name="s")
    return pl.kernel(
        _seg_body,
        out_type=[jax.ShapeDtypeStruct((_NC, _NPAD, _D), jnp.float32)],
        mesh=mesh,
        scratch_types=(
            [pltpu.VMEM((_G, _CH), jnp.int32)] * 4
            + [pltpu.VMEM((_CH, _D), jnp.float32)] * _NBUF
            + [pltpu.VMEM_SHARED((_NPAD, _D), jnp.float32)]
            + [pltpu.SemaphoreType.DMA] * (2 * _NBUF + 1)
        ),
        compiler_params=pltpu.CompilerParams(use_tc_tiling_on_sc=False),
        name="seg_sum",
    )


def _make_deg():
    mesh = plsc.VectorSubcoreMesh(core_axis_name="c", subcore_axis_name="s")
    return pl.kernel(
        _deg_body,
        out_type=[jax.ShapeDtypeStruct((_NC, _NPAD, 16), jnp.float32)],
        mesh=mesh,
        scratch_types=[
            pltpu.VMEM((_NCH32, _DCH), jnp.int32),
            pltpu.VMEM((128, 16), jnp.float32),
            pltpu.VMEM_SHARED((_NPAD, 16), jnp.float32),
        ],
        compiler_params=pltpu.CompilerParams(use_tc_tiling_on_sc=False),
        name="deg_sum",
    )


_seg = _make_seg()
_deg = _make_deg()


def _bn(h, g, be):
    m = jnp.mean(h, axis=0, keepdims=True)
    v = jnp.mean((h - m) * (h - m), axis=0, keepdims=True)
    return (h - m) / jnp.sqrt(v + 1e-5) * g + be


def _deg_from(dp_ref):
    deg = dp_ref[0, : _N, 0:1] + dp_ref[1, : _N, 0:1]
    return jnp.maximum(deg, 1.0)


def _tc1_body(sp_ref, dp_ref, x_ref, wl1t_ref, wr1t_ref, b1_ref, g1_ref,
              be1_ref, h1_ref):
    s = sp_ref[0, : _N, :] + sp_ref[1, : _N, :]
    agg = s / _deg_from(dp_ref)
    pre = (jnp.dot(agg, wl1t_ref[...], preferred_element_type=jnp.float32)
           + b1_ref[...]
           + jnp.dot(x_ref[...], wr1t_ref[...],
                     preferred_element_type=jnp.float32))
    h = jnp.maximum(pre, 0.0)
    h1_ref[...] = _bn(h, g1_ref[...], be1_ref[...])


def _tc2_body(sp_ref, dp_ref, h1_ref, wl2t_ref, wr2t_ref, b2_ref, g2_ref,
              be2_ref, wl3t_ref, h2_ref, y3_ref):
    s = sp_ref[0, : _N, :] + sp_ref[1, : _N, :]
    agg = s / _deg_from(dp_ref)
    pre = (jnp.dot(agg, wl2t_ref[...], preferred_element_type=jnp.float32)
           + b2_ref[...]
           + jnp.dot(h1_ref[...], wr2t_ref[...],
                     preferred_element_type=jnp.float32))
    h = jnp.maximum(pre, 0.0)
    h2 = _bn(h, g2_ref[...], be2_ref[...])
    h2_ref[...] = h2
    y3_ref[...] = jnp.dot(h2, wl3t_ref[...],
                          preferred_element_type=jnp.float32)


def _tc3_body(sp_ref, dp_ref, h2_ref, wr3t_ref, b3_ref, eps_ref, z_ref):
    s = sp_ref[0, : _N, :] + sp_ref[1, : _N, :]
    pre = (s / _deg_from(dp_ref) + b3_ref[...]
           + jnp.dot(h2_ref[...], wr3t_ref[...],
                     preferred_element_type=jnp.float32))
    mean = pre[:, : 64]
    log_std = pre[:, 64:]
    z_ref[...] = mean + jnp.exp(log_std) * eps_ref[...]


_tc1 = pl.pallas_call(
    _tc1_body, out_shape=jax.ShapeDtypeStruct((_N, _D), jnp.float32))
_tc2 = pl.pallas_call(
    _tc2_body, out_shape=[jax.ShapeDtypeStruct((_N, 256), jnp.float32),
                          jax.ShapeDtypeStruct((_N, _D), jnp.float32)])
_tc3 = pl.pallas_call(
    _tc3_body, out_shape=jax.ShapeDtypeStruct((_N, 64), jnp.float32))


def kernel(x, edge_index, Wl1, Wr1, b1, g1, be1, Wl2, Wr2, b2, g2, be2,
           Wl3, Wr3, b3, eps):
    src = edge_index[0].reshape(_NW, _NCHUNK, _CH)
    dst = edge_index[1].reshape(_NW, _NCHUNK, _CH)
    dst32 = edge_index[1].reshape(_NW, _NCH32, _DCH)

    (d1,) = _deg(dst32)
    (s1,) = _seg(x, src, dst)
    h1 = _tc1(s1, d1, x, Wl1.T, Wr1.T, b1[None, :], g1[None, :],
              be1[None, :])
    (s2,) = _seg(h1, src, dst)
    h2, y3 = _tc2(s2, d1, h1, Wl2.T, Wr2.T, b2[None, :], g2[None, :],
                  be2[None, :], Wl3.T)
    (s3,) = _seg(y3, src, dst)
    z = _tc3(s3, d1, h2, Wr3.T, b3[None, :], eps)
    return z


# speedup vs baseline: 1.4925x; 1.1031x over previous
import functools

import jax
import jax.numpy as jnp
from jax import lax
from jax.experimental import pallas as pl
from jax.experimental.pallas import tpu as pltpu
from jax.experimental.pallas import tpu_sc as plsc

_N = 10000
_E = 320000
_D = 128
_NC = 2
_NS = 16
_NW = _NC * _NS
_EPT = _E // _NW
_CH = 125
_NCHUNK = _EPT // _CH
_NPAD = 10000
_RPT = _NPAD // _NS
_NBUF = 3
_G = 5
_NGRP = _NCHUNK // _G
_DCH = 125
_NCH32 = _EPT // _DCH


def _seg_body(y_hbm, src_hbm, dst_hbm, agg_out, srcA, dstA, srcB, dstB,
              rows0, rows1, rows2, acc, semg0, semg1, semg2,
              sems0, sems1, sems2, semi):
    c = lax.axis_index("c")
    s = lax.axis_index("s")
    blk = c * _NS + s
    rows = [rows0, rows1, rows2]
    semg = [semg0, semg1, semg2]
    sems = [sems0, sems1, sems2]
    idx = [(srcA, dstA), (srcB, dstB)]

    def _zrow(i, _):
        for k in range(8):
            rows0[i, pl.ds(k * 16, 16)] = jnp.zeros((16,), jnp.float32)
        return 0
    lax.fori_loop(0, _CH, _zrow, 0)
    base = s * _RPT
    for t in range(_RPT // _CH):
        pltpu.sync_copy(rows0, acc.at[pl.ds(base + t * _CH, _CH)])
    plsc.subcore_barrier()

    pltpu.sync_copy(src_hbm.at[blk, pl.ds(0, _G)], srcA)
    pltpu.sync_copy(dst_hbm.at[blk, pl.ds(0, _G)], dstA)
    cpg = [None] * _NCHUNK
    cps = [None] * _NCHUNK
    cpi = None

    def _gather(j):
        g, r = divmod(j, _G)
        srcg = idx[g % 2][0]
        cpg[j] = pltpu.async_copy(y_hbm.at[srcg.at[r]], rows[j % _NBUF],
                                  semg[j % _NBUF])

    _gather(0)
    _gather(1)
    for j in range(_NCHUNK):
        b = j % _NBUF
        g, r = divmod(j, _G)
        if r == 2 and g + 1 < _NGRP:
            srcn, dstn = idx[(g + 1) % 2]
            cpi = (pltpu.async_copy(src_hbm.at[blk, pl.ds((g + 1) * _G, _G)],
                                    srcn, semi),
                   pltpu.async_copy(dst_hbm.at[blk, pl.ds((g + 1) * _G, _G)],
                                    dstn, semi))
        if r == 3 and g + 1 < _NGRP:
            cpi[0].wait()
            cpi[1].wait()
        cpg[j].wait()
        dstg = idx[g % 2][1]
        cps[j] = pltpu.async_copy(rows[b], acc.at[dstg.at[r]], sems[b],
                                  add=True)
        if j >= 1:
            cps[j - 1].wait()
        if j + 2 < _NCHUNK:
            _gather(j + 2)
    cps[_NCHUNK - 1].wait()

    plsc.subcore_barrier()

    for t in range(_RPT // _CH):
        r = base + t * _CH
        pltpu.sync_copy(acc.at[pl.ds(r, _CH)], rows0)
        pltpu.sync_copy(rows0, agg_out.at[c, pl.ds(r, _CH)])


def _deg_body(dst_hbm, deg_out, dstv, ones, dacc):
    c = lax.axis_index("c")
    s = lax.axis_index("s")
    blk = c * _NS + s

    pltpu.sync_copy(dst_hbm.at[blk], dstv)

    def _fill(val):
        def _row(i, _):
            ones[i, :] = jnp.full((16,), val, jnp.float32)
            return 0
        lax.fori_loop(0, _DCH, _row, 0)
    _fill(0.0)
    for t in range(_RPT // _DCH):
        pltpu.sync_copy(ones, dacc.at[pl.ds(s * _RPT + t * _DCH, _DCH)])
    _fill(1.0)
    plsc.subcore_barrier()

    def _step(j, _):
        pltpu.sync_copy(ones, dacc.at[dstv.at[j]], add=True)
        return 0
    lax.fori_loop(0, _NCH32, _step, 0)

    plsc.subcore_barrier()

    for t in range(_RPT // _DCH):
        r = s * _RPT + t * _DCH
        pltpu.sync_copy(dacc.at[pl.ds(r, _DCH)], ones)
        pltpu.sync_copy(ones, deg_out.at[c, pl.ds(r, _DCH)])


def _make_seg():
    mesh = plsc.VectorSubcoreMesh(core_axis_name="c", subcore_axis_name="s")
    return pl.kernel(
        _seg_body,
        out_type=[jax.ShapeDtypeStruct((_NC, _NPAD, _D), jnp.float32)],
        mesh=mesh,
        scratch_types=(
            [pltpu.VMEM((_G, _CH), jnp.int32)] * 4
            + [pltpu.VMEM((_CH, _D), jnp.float32)] * _NBUF
            + [pltpu.VMEM_SHARED((_NPAD, _D), jnp.float32)]
            + [pltpu.SemaphoreType.DMA] * (2 * _NBUF + 1)
        ),
        compiler_params=pltpu.CompilerParams(use_tc_tiling_on_sc=False),
        name="seg_sum",
    )


def _make_deg():
    mesh = plsc.VectorSubcoreMesh(core_axis_name="c", subcore_axis_name="s")
    return pl.kernel(
        _deg_body,
        out_type=[jax.ShapeDtypeStruct((_NC, _NPAD, 16), jnp.float32)],
        mesh=mesh,
        scratch_types=[
            pltpu.VMEM((_NCH32, _DCH), jnp.int32),
            pltpu.VMEM((_DCH, 16), jnp.float32),
            pltpu.VMEM_SHARED((_NPAD, 16), jnp.float32),
        ],
        compiler_params=pltpu.CompilerParams(use_tc_tiling_on_sc=False),
        name="deg_sum",
    )


_seg = _make_seg()
_deg = _make_deg()


def _bn(h, g, be):
    m = jnp.mean(h, axis=0, keepdims=True)
    v = jnp.mean((h - m) * (h - m), axis=0, keepdims=True)
    return (h - m) / jnp.sqrt(v + 1e-5) * g + be


def _deg_from(dp_ref):
    deg = dp_ref[0, : _N, 0:1] + dp_ref[1, : _N, 0:1]
    return jnp.maximum(deg, 1.0)


def _tc1_body(sp_ref, dp_ref, x_ref, wl1t_ref, wr1t_ref, b1_ref, g1_ref,
              be1_ref, h1_ref):
    s = sp_ref[0, : _N, :] + sp_ref[1, : _N, :]
    agg = s / _deg_from(dp_ref)
    pre = (jnp.dot(agg, wl1t_ref[...], preferred_element_type=jnp.float32)
           + b1_ref[...]
           + jnp.dot(x_ref[...], wr1t_ref[...],
                     preferred_element_type=jnp.float32))
    h = jnp.maximum(pre, 0.0)
    h1_ref[...] = _bn(h, g1_ref[...], be1_ref[...])


def _tc2_body(sp_ref, dp_ref, h1_ref, wl2t_ref, wr2t_ref, b2_ref, g2_ref,
              be2_ref, wl3t_ref, h2_ref, y3_ref):
    s = sp_ref[0, : _N, :] + sp_ref[1, : _N, :]
    agg = s / _deg_from(dp_ref)
    pre = (jnp.dot(agg, wl2t_ref[...], preferred_element_type=jnp.float32)
           + b2_ref[...]
           + jnp.dot(h1_ref[...], wr2t_ref[...],
                     preferred_element_type=jnp.float32))
    h = jnp.maximum(pre, 0.0)
    h2 = _bn(h, g2_ref[...], be2_ref[...])
    h2_ref[...] = h2
    y3_ref[...] = jnp.dot(h2, wl3t_ref[...],
                          preferred_element_type=jnp.float32)


def _tc3_body(sp_ref, dp_ref, h2_ref, wr3t_ref, b3_ref, eps_ref, z_ref):
    s = sp_ref[0, : _N, :] + sp_ref[1, : _N, :]
    pre = (s / _deg_from(dp_ref) + b3_ref[...]
           + jnp.dot(h2_ref[...], wr3t_ref[...],
                     preferred_element_type=jnp.float32))
    mean = pre[:, : 64]
    log_std = pre[:, 64:]
    z_ref[...] = mean + jnp.exp(log_std) * eps_ref[...]


_tc1 = pl.pallas_call(
    _tc1_body, out_shape=jax.ShapeDtypeStruct((_N, _D), jnp.float32))
_tc2 = pl.pallas_call(
    _tc2_body, out_shape=[jax.ShapeDtypeStruct((_N, 256), jnp.float32),
                          jax.ShapeDtypeStruct((_N, _D), jnp.float32)])
_tc3 = pl.pallas_call(
    _tc3_body, out_shape=jax.ShapeDtypeStruct((_N, 64), jnp.float32))


def kernel(x, edge_index, Wl1, Wr1, b1, g1, be1, Wl2, Wr2, b2, g2, be2,
           Wl3, Wr3, b3, eps):
    src = edge_index[0].reshape(_NW, _NCHUNK, _CH)
    dst = edge_index[1].reshape(_NW, _NCHUNK, _CH)
    dst32 = edge_index[1].reshape(_NW, _NCH32, _DCH)

    (d1,) = _deg(dst32)
    (s1,) = _seg(x, src, dst)
    h1 = _tc1(s1, d1, x, Wl1.T, Wr1.T, b1[None, :], g1[None, :],
              be1[None, :])
    (s2,) = _seg(h1, src, dst)
    h2, y3 = _tc2(s2, d1, h1, Wl2.T, Wr2.T, b2[None, :], g2[None, :],
                  be2[None, :], Wl3.T)
    (s3,) = _seg(y3, src, dst)
    z = _tc3(s3, d1, h2, Wr3.T, b3[None, :], eps)
    return z


# speedup vs baseline: 1.5102x; 1.0118x over previous
import functools

import jax
import jax.numpy as jnp
from jax import lax
from jax.experimental import pallas as pl
from jax.experimental.pallas import tpu as pltpu
from jax.experimental.pallas import tpu_sc as plsc

_N = 10000
_E = 320000
_D = 128
_NC = 2
_NS = 16
_NW = _NC * _NS
_EPT = _E // _NW
_CH = 125
_NCHUNK = _EPT // _CH
_NPAD = 10000
_RPT = _NPAD // _NS
_NBUF = 3
_G = 5
_NGRP = _NCHUNK // _G
_DCH = 125
_NCH32 = _EPT // _DCH


def _seg_body(y_hbm, src_hbm, dst_hbm, agg_out, srcA, dstA, srcB, dstB,
              rows0, rows1, rows2, acc, semg0, semg1, semg2,
              sems0, sems1, sems2, semi):
    c = lax.axis_index("c")
    s = lax.axis_index("s")
    blk = c * _NS + s
    rows = [rows0, rows1, rows2]
    semg = [semg0, semg1, semg2]
    sems = [sems0, sems1, sems2]
    idx = [(srcA, dstA), (srcB, dstB)]

    def _zrow(i, _):
        for k in range(8):
            rows0[i, pl.ds(k * 16, 16)] = jnp.zeros((16,), jnp.float32)
        return 0
    lax.fori_loop(0, _CH, _zrow, 0)
    base = s * _RPT
    cpz = [pltpu.async_copy(rows0, acc.at[pl.ds(base + t * _CH, _CH)],
                            semg0)
           for t in range(_RPT // _CH)]
    for cp in cpz:
        cp.wait()
    plsc.subcore_barrier()

    pltpu.sync_copy(src_hbm.at[blk, pl.ds(0, _G)], srcA)
    pltpu.sync_copy(dst_hbm.at[blk, pl.ds(0, _G)], dstA)
    cpg = [None] * _NCHUNK
    cps = [None] * _NCHUNK
    cpi = None

    def _gather(j):
        g, r = divmod(j, _G)
        srcg = idx[g % 2][0]
        cpg[j] = pltpu.async_copy(y_hbm.at[srcg.at[r]], rows[j % _NBUF],
                                  semg[j % _NBUF])

    _gather(0)
    _gather(1)
    for j in range(_NCHUNK):
        b = j % _NBUF
        g, r = divmod(j, _G)
        if r == 2 and g + 1 < _NGRP:
            srcn, dstn = idx[(g + 1) % 2]
            cpi = (pltpu.async_copy(src_hbm.at[blk, pl.ds((g + 1) * _G, _G)],
                                    srcn, semi),
                   pltpu.async_copy(dst_hbm.at[blk, pl.ds((g + 1) * _G, _G)],
                                    dstn, semi))
        if r == 3 and g + 1 < _NGRP:
            cpi[0].wait()
            cpi[1].wait()
        cpg[j].wait()
        dstg = idx[g % 2][1]
        cps[j] = pltpu.async_copy(rows[b], acc.at[dstg.at[r]], sems[b],
                                  add=True)
        if j >= 1:
            cps[j - 1].wait()
        if j + 2 < _NCHUNK:
            _gather(j + 2)
    cps[_NCHUNK - 1].wait()

    plsc.subcore_barrier()

    nt = _RPT // _CH
    cpo = [None] * nt
    for t in range(nt):
        b = t % _NBUF
        if t >= _NBUF:
            cpo[t - _NBUF].wait()
        r = base + t * _CH
        pltpu.async_copy(acc.at[pl.ds(r, _CH)], rows[b], semg[b]).wait()
        cpo[t] = pltpu.async_copy(rows[b], agg_out.at[c, pl.ds(r, _CH)],
                                  sems[b])
    for t in range(max(0, nt - _NBUF), nt):
        cpo[t].wait()


def _deg_body(dst_hbm, deg_out, dstv, ones, dacc):
    c = lax.axis_index("c")
    s = lax.axis_index("s")
    blk = c * _NS + s

    pltpu.sync_copy(dst_hbm.at[blk], dstv)

    def _fill(val):
        def _row(i, _):
            ones[i, :] = jnp.full((16,), val, jnp.float32)
            return 0
        lax.fori_loop(0, _DCH, _row, 0)
    _fill(0.0)
    for t in range(_RPT // _DCH):
        pltpu.sync_copy(ones, dacc.at[pl.ds(s * _RPT + t * _DCH, _DCH)])
    _fill(1.0)
    plsc.subcore_barrier()

    def _step(j, _):
        pltpu.sync_copy(ones, dacc.at[dstv.at[j]], add=True)
        return 0
    lax.fori_loop(0, _NCH32, _step, 0)

    plsc.subcore_barrier()

    for t in range(_RPT // _DCH):
        r = s * _RPT + t * _DCH
        pltpu.sync_copy(dacc.at[pl.ds(r, _DCH)], ones)
        pltpu.sync_copy(ones, deg_out.at[c, pl.ds(r, _DCH)])


def _make_seg():
    mesh = plsc.VectorSubcoreMesh(core_axis_name="c", subcore_axis_name="s")
    return pl.kernel(
        _seg_body,
        out_type=[jax.ShapeDtypeStruct((_NC, _NPAD, _D), jnp.float32)],
        mesh=mesh,
        scratch_types=(
            [pltpu.VMEM((_G, _CH), jnp.int32)] * 4
            + [pltpu.VMEM((_CH, _D), jnp.float32)] * _NBUF
            + [pltpu.VMEM_SHARED((_NPAD, _D), jnp.float32)]
            + [pltpu.SemaphoreType.DMA] * (2 * _NBUF + 1)
        ),
        compiler_params=pltpu.CompilerParams(use_tc_tiling_on_sc=False),
        name="seg_sum",
    )


def _make_deg():
    mesh = plsc.VectorSubcoreMesh(core_axis_name="c", subcore_axis_name="s")
    return pl.kernel(
        _deg_body,
        out_type=[jax.ShapeDtypeStruct((_NC, _NPAD, 16), jnp.float32)],
        mesh=mesh,
        scratch_types=[
            pltpu.VMEM((_NCH32, _DCH), jnp.int32),
            pltpu.VMEM((_DCH, 16), jnp.float32),
            pltpu.VMEM_SHARED((_NPAD, 16), jnp.float32),
        ],
        compiler_params=pltpu.CompilerParams(use_tc_tiling_on_sc=False),
        name="deg_sum",
    )


_seg = _make_seg()
_deg = _make_deg()


def _bn(h, g, be):
    m = jnp.mean(h, axis=0, keepdims=True)
    v = jnp.mean((h - m) * (h - m), axis=0, keepdims=True)
    return (h - m) / jnp.sqrt(v + 1e-5) * g + be


def _deg_from(dp_ref):
    deg = dp_ref[0, : _N, 0:1] + dp_ref[1, : _N, 0:1]
    return jnp.maximum(deg, 1.0)


def _tc1_body(sp_ref, dp_ref, x_ref, wl1t_ref, wr1t_ref, b1_ref, g1_ref,
              be1_ref, h1_ref):
    s = sp_ref[0, : _N, :] + sp_ref[1, : _N, :]
    agg = s / _deg_from(dp_ref)
    pre = (jnp.dot(agg, wl1t_ref[...], preferred_element_type=jnp.float32)
           + b1_ref[...]
           + jnp.dot(x_ref[...], wr1t_ref[...],
                     preferred_element_type=jnp.float32))
    h = jnp.maximum(pre, 0.0)
    h1_ref[...] = _bn(h, g1_ref[...], be1_ref[...])


def _tc2_body(sp_ref, dp_ref, h1_ref, wl2t_ref, wr2t_ref, b2_ref, g2_ref,
              be2_ref, wl3t_ref, h2_ref, y3_ref):
    s = sp_ref[0, : _N, :] + sp_ref[1, : _N, :]
    agg = s / _deg_from(dp_ref)
    pre = (jnp.dot(agg, wl2t_ref[...], preferred_element_type=jnp.float32)
           + b2_ref[...]
           + jnp.dot(h1_ref[...], wr2t_ref[...],
                     preferred_element_type=jnp.float32))
    h = jnp.maximum(pre, 0.0)
    h2 = _bn(h, g2_ref[...], be2_ref[...])
    h2_ref[...] = h2
    y3_ref[...] = jnp.dot(h2, wl3t_ref[...],
                          preferred_element_type=jnp.float32)


def _tc3_body(sp_ref, dp_ref, h2_ref, wr3t_ref, b3_ref, eps_ref, z_ref):
    s = sp_ref[0, : _N, :] + sp_ref[1, : _N, :]
    pre = (s / _deg_from(dp_ref) + b3_ref[...]
           + jnp.dot(h2_ref[...], wr3t_ref[...],
                     preferred_element_type=jnp.float32))
    mean = pre[:, : 64]
    log_std = pre[:, 64:]
    z_ref[...] = mean + jnp.exp(log_std) * eps_ref[...]


_tc1 = pl.pallas_call(
    _tc1_body, out_shape=jax.ShapeDtypeStruct((_N, _D), jnp.float32))
_tc2 = pl.pallas_call(
    _tc2_body, out_shape=[jax.ShapeDtypeStruct((_N, 256), jnp.float32),
                          jax.ShapeDtypeStruct((_N, _D), jnp.float32)])
_tc3 = pl.pallas_call(
    _tc3_body, out_shape=jax.ShapeDtypeStruct((_N, 64), jnp.float32))


def kernel(x, edge_index, Wl1, Wr1, b1, g1, be1, Wl2, Wr2, b2, g2, be2,
           Wl3, Wr3, b3, eps):
    src = edge_index[0].reshape(_NW, _NCHUNK, _CH)
    dst = edge_index[1].reshape(_NW, _NCHUNK, _CH)
    dst32 = edge_index[1].reshape(_NW, _NCH32, _DCH)

    (d1,) = _deg(dst32)
    (s1,) = _seg(x, src, dst)
    h1 = _tc1(s1, d1, x, Wl1.T, Wr1.T, b1[None, :], g1[None, :],
              be1[None, :])
    (s2,) = _seg(h1, src, dst)
    h2, y3 = _tc2(s2, d1, h1, Wl2.T, Wr2.T, b2[None, :], g2[None, :],
                  be2[None, :], Wl3.T)
    (s3,) = _seg(y3, src, dst)
    z = _tc3(s3, d1, h2, Wr3.T, b3[None, :], eps)
    return z


# speedup vs baseline: 1.5161x; 1.0039x over previous
import jax
import jax.numpy as jnp
from jax import lax
from jax.experimental import pallas as pl
from jax.experimental.pallas import tpu as pltpu
from jax.experimental.pallas import tpu_sc as plsc

_N = 10000
_E = 320000
_D = 128
_NC = 2
_NS = 16
_NW = _NC * _NS
_EPT = _E // _NW
_CH = 125
_NCHUNK = _EPT // _CH
_NPAD = 10000
_RPT = _NPAD // _NS
_NBUF = 3
_G = 5
_NGRP = _NCHUNK // _G
_DCH = 125
_NCH32 = _EPT // _DCH


def _seg_body(y_hbm, src_hbm, dst_hbm, agg_out, srcA, dstA, srcB, dstB,
              rows0, rows1, rows2, acc, semg0, semg1, semg2,
              sems0, sems1, sems2, semi):
    c = lax.axis_index("c")
    s = lax.axis_index("s")
    blk = c * _NS + s
    rows = [rows0, rows1, rows2]
    semg = [semg0, semg1, semg2]
    sems = [sems0, sems1, sems2]
    idx = [(srcA, dstA), (srcB, dstB)]

    def _zrow(i, _):
        for k in range(8):
            rows0[i, pl.ds(k * 16, 16)] = jnp.zeros((16,), jnp.float32)
        return 0
    lax.fori_loop(0, _CH, _zrow, 0)
    base = s * _RPT
    cpz = [pltpu.async_copy(rows0, acc.at[pl.ds(base + t * _CH, _CH)],
                            semg0)
           for t in range(_RPT // _CH)]
    for cp in cpz:
        cp.wait()
    plsc.subcore_barrier()

    pltpu.sync_copy(src_hbm.at[blk, pl.ds(0, _G)], srcA)
    pltpu.sync_copy(dst_hbm.at[blk, pl.ds(0, _G)], dstA)
    cpg = [None] * _NCHUNK
    cps = [None] * _NCHUNK
    cpi = None

    def _gather(j):
        g, r = divmod(j, _G)
        srcg = idx[g % 2][0]
        cpg[j] = pltpu.async_copy(y_hbm.at[srcg.at[r]], rows[j % _NBUF],
                                  semg[j % _NBUF])

    _gather(0)
    _gather(1)
    for j in range(_NCHUNK):
        b = j % _NBUF
        g, r = divmod(j, _G)
        if r == 2 and g + 1 < _NGRP:
            srcn, dstn = idx[(g + 1) % 2]
            cpi = (pltpu.async_copy(src_hbm.at[blk, pl.ds((g + 1) * _G, _G)],
                                    srcn, semi),
                   pltpu.async_copy(dst_hbm.at[blk, pl.ds((g + 1) * _G, _G)],
                                    dstn, semi))
        if r == 3 and g + 1 < _NGRP:
            cpi[0].wait()
            cpi[1].wait()
        cpg[j].wait()
        dstg = idx[g % 2][1]
        cps[j] = pltpu.async_copy(rows[b], acc.at[dstg.at[r]], sems[b],
                                  add=True)
        if j >= 1:
            cps[j - 1].wait()
        if j + 2 < _NCHUNK:
            _gather(j + 2)
    cps[_NCHUNK - 1].wait()

    plsc.subcore_barrier()

    nt = _RPT // _CH
    cpo = [None] * nt
    for t in range(nt):
        b = t % _NBUF
        if t >= _NBUF:
            cpo[t - _NBUF].wait()
        r = base + t * _CH
        pltpu.async_copy(acc.at[pl.ds(r, _CH)], rows[b], semg[b]).wait()
        cpo[t] = pltpu.async_copy(rows[b], agg_out.at[c, pl.ds(r, _CH)],
                                  sems[b])
    for t in range(max(0, nt - _NBUF), nt):
        cpo[t].wait()


def _deg_body(dst_hbm, deg_out, dstv, ones, dacc):
    c = lax.axis_index("c")
    s = lax.axis_index("s")
    blk = c * _NS + s

    pltpu.sync_copy(dst_hbm.at[blk], dstv)

    def _fill(val):
        def _row(i, _):
            ones[i, :] = jnp.full((16,), val, jnp.float32)
            return 0
        lax.fori_loop(0, _DCH, _row, 0)
    _fill(0.0)
    for t in range(_RPT // _DCH):
        pltpu.sync_copy(ones, dacc.at[pl.ds(s * _RPT + t * _DCH, _DCH)])
    _fill(1.0)
    plsc.subcore_barrier()

    def _step(j, _):
        pltpu.sync_copy(ones, dacc.at[dstv.at[j]], add=True)
        return 0
    lax.fori_loop(0, _NCH32, _step, 0)

    plsc.subcore_barrier()

    for t in range(_RPT // _DCH):
        r = s * _RPT + t * _DCH
        pltpu.sync_copy(dacc.at[pl.ds(r, _DCH)], ones)
        pltpu.sync_copy(ones, deg_out.at[c, pl.ds(r, _DCH)])


def _make_seg():
    mesh = plsc.VectorSubcoreMesh(core_axis_name="c", subcore_axis_name="s")
    return pl.kernel(
        _seg_body,
        out_type=[jax.ShapeDtypeStruct((_NC, _NPAD, _D), jnp.float32)],
        mesh=mesh,
        scratch_types=(
            [pltpu.VMEM((_G, _CH), jnp.int32)] * 4
            + [pltpu.VMEM((_CH, _D), jnp.float32)] * _NBUF
            + [pltpu.VMEM_SHARED((_NPAD, _D), jnp.float32)]
            + [pltpu.SemaphoreType.DMA] * (2 * _NBUF + 1)
        ),
        compiler_params=pltpu.CompilerParams(use_tc_tiling_on_sc=False),
        name="seg_sum",
    )


def _make_deg():
    mesh = plsc.VectorSubcoreMesh(core_axis_name="c", subcore_axis_name="s")
    return pl.kernel(
        _deg_body,
        out_type=[jax.ShapeDtypeStruct((_NC, _NPAD, 16), jnp.float32)],
        mesh=mesh,
        scratch_types=[
            pltpu.VMEM((_NCH32, _DCH), jnp.int32),
            pltpu.VMEM((_DCH, 16), jnp.float32),
            pltpu.VMEM_SHARED((_NPAD, 16), jnp.float32),
        ],
        compiler_params=pltpu.CompilerParams(use_tc_tiling_on_sc=False),
        name="deg_sum",
    )


_seg = _make_seg()
_deg = _make_deg()


def _bn(h, g, be):
    m = jnp.mean(h, axis=0, keepdims=True)
    v = jnp.mean((h - m) * (h - m), axis=0, keepdims=True)
    return (h - m) / jnp.sqrt(v + 1e-5) * g + be


def _deg_from(dp_ref):
    deg = dp_ref[0, : _N, 0:1] + dp_ref[1, : _N, 0:1]
    return jnp.maximum(deg, 1.0)


def _tc1_body(sp_ref, dp_ref, x_ref, wl1t_ref, wr1t_ref, b1_ref, g1_ref,
              be1_ref, h1_ref):
    s = sp_ref[0, : _N, :] + sp_ref[1, : _N, :]
    agg = s / _deg_from(dp_ref)
    pre = (jnp.dot(agg, wl1t_ref[...], preferred_element_type=jnp.float32)
           + b1_ref[...]
           + jnp.dot(x_ref[...], wr1t_ref[...],
                     preferred_element_type=jnp.float32))
    h = jnp.maximum(pre, 0.0)
    h1_ref[...] = _bn(h, g1_ref[...], be1_ref[...])


def _tc2_body(sp_ref, dp_ref, h1_ref, wl2t_ref, wr2t_ref, b2_ref, g2_ref,
              be2_ref, wl3t_ref, h2_ref, y3_ref):
    s = sp_ref[0, : _N, :] + sp_ref[1, : _N, :]
    agg = s / _deg_from(dp_ref)
    pre = (jnp.dot(agg, wl2t_ref[...], preferred_element_type=jnp.float32)
           + b2_ref[...]
           + jnp.dot(h1_ref[...], wr2t_ref[...],
                     preferred_element_type=jnp.float32))
    h = jnp.maximum(pre, 0.0)
    h2 = _bn(h, g2_ref[...], be2_ref[...])
    h2_ref[...] = h2
    y3_ref[...] = jnp.dot(h2, wl3t_ref[...],
                          preferred_element_type=jnp.float32)


def _tc3_body(sp_ref, dp_ref, h2_ref, wr3t_ref, b3_ref, eps_ref, z_ref):
    s = sp_ref[0, : _N, :] + sp_ref[1, : _N, :]
    pre = (s / _deg_from(dp_ref) + b3_ref[...]
           + jnp.dot(h2_ref[...], wr3t_ref[...],
                     preferred_element_type=jnp.float32))
    mean = pre[:, : 64]
    log_std = pre[:, 64:]
    z_ref[...] = mean + jnp.exp(log_std) * eps_ref[...]


_tc1 = pl.pallas_call(
    _tc1_body, out_shape=jax.ShapeDtypeStruct((_N, _D), jnp.float32))
_tc2 = pl.pallas_call(
    _tc2_body, out_shape=[jax.ShapeDtypeStruct((_N, 256), jnp.float32),
                          jax.ShapeDtypeStruct((_N, _D), jnp.float32)])
_tc3 = pl.pallas_call(
    _tc3_body, out_shape=jax.ShapeDtypeStruct((_N, 64), jnp.float32))


def kernel(x, edge_index, Wl1, Wr1, b1, g1, be1, Wl2, Wr2, b2, g2, be2,
           Wl3, Wr3, b3, eps):
    src = edge_index[0].reshape(_NW, _NCHUNK, _CH)
    dst = edge_index[1].reshape(_NW, _NCHUNK, _CH)
    dst32 = edge_index[1].reshape(_NW, _NCH32, _DCH)

    (d1,) = _deg(dst32)
    (s1,) = _seg(x, src, dst)
    h1 = _tc1(s1, d1, x, Wl1.T, Wr1.T, b1[None, :], g1[None, :],
              be1[None, :])
    (s2,) = _seg(h1, src, dst)
    h2, y3 = _tc2(s2, d1, h1, Wl2.T, Wr2.T, b2[None, :], g2[None, :],
                  be2[None, :], Wl3.T)
    (s3,) = _seg(y3, src, dst)
    z = _tc3(s3, d1, h2, Wr3.T, b3[None, :], eps)
    return z
